# Initial kernel scaffold; baseline (speedup 1.0000x reference)
import jax
import jax.numpy as jnp
from jax.experimental import pallas as pl


def kernel(x, edge_index, W1_l, W1_r, b1, W2_l, W2_r, b2):
    raise NotImplementedError("write your pallas kernel here")


# speedup vs baseline: 5.7187x; 5.7187x over previous
import jax
import jax.numpy as jnp
from jax import lax
from jax.experimental import pallas as pl
from jax.experimental.pallas import tpu as pltpu
from jax.experimental.pallas import tpu_sc as plsc

N = 10000
N2 = 10240
E = 160000
E2 = 163840
DIN = 256
DHID = 512
DOUT = 256
HALF = 128
NT = 16
_f32 = jnp.float32

CH = 64
NB = 8
NCHUNK = E2 // (NT * CH)
NBLK = NCHUNK // NB
RPT = N2 // NT


def _sc_agg_body(table, srcs, dsts, agg_out, accum, sidx, didx,
                 gbuf0, gbuf1, gsem0, gsem1):
    c = lax.axis_index("c")
    s = lax.axis_index("s")
    base = s * RPT

    def _zrow(i, _):
        for k in range(HALF // 16):
            gbuf0[i, pl.ds(k * 16, 16)] = jnp.zeros((16,), _f32)
        return 0
    lax.fori_loop(0, CH, _zrow, 0)
    for q in range(RPT // CH):
        pltpu.sync_copy(gbuf0, accum.at[pl.ds(base + q * CH, CH)])

    plsc.subcore_barrier()

    def _blk(b, _):
        pltpu.sync_copy(srcs.at[s, pl.ds(b * NB, NB)], sidx)
        pltpu.sync_copy(dsts.at[s, pl.ds(b * NB, NB)], didx)

        def _fix(j, _):
            for k in range(CH // 16):
                sl = sidx[j, pl.ds(k * 16, 16)]
                sidx[j, pl.ds(k * 16, 16)] = sl * 2 + c
            return 0
        lax.fori_loop(0, NB, _fix, 0)

        bufs = (gbuf0, gbuf1)
        sems = (gsem0, gsem1)
        pltpu.async_copy(table.at[sidx.at[0]], bufs[0], sems[0])
        for j in range(NB):
            bj, sj = bufs[j % 2], sems[j % 2]
            pltpu.make_async_copy(table.at[sidx.at[j]], bj, sj).wait()
            if j + 1 < NB:
                pltpu.async_copy(table.at[sidx.at[j + 1]],
                                 bufs[(j + 1) % 2], sems[(j + 1) % 2])
            pltpu.sync_copy(bj, accum.at[didx.at[j]], add=True)
        return 0
    lax.fori_loop(0, NBLK, _blk, 0)

    plsc.subcore_barrier()

    pltpu.sync_copy(accum.at[pl.ds(base, RPT)],
                    agg_out.at[pl.ds(base, RPT), c])


def _sc_deg_body(dsts, deg_out, deg2d, didx, ones_v, dsem):
    c = lax.axis_index("c")
    s = lax.axis_index("s")
    base = s * RPT

    def _zrow(i, _):
        for k in range(HALF // 16):
            ones_v[i, pl.ds(k * 16, 16)] = jnp.zeros((16,), _f32)
        return 0
    lax.fori_loop(0, CH, _zrow, 0)
    for q in range(RPT // CH):
        pltpu.sync_copy(ones_v, deg2d.at[pl.ds(base + q * CH, CH)])

    def _orow(i, _):
        for k in range(HALF // 16):
            ones_v[i, pl.ds(k * 16, 16)] = jnp.full((16,), 1.0, _f32)
        return 0
    lax.fori_loop(0, CH, _orow, 0)

    plsc.subcore_barrier()

    def _blk(b, _):
        pltpu.sync_copy(dsts.at[s, pl.ds(b * NB, NB)], didx)
        for j in range(NB):
            pltpu.async_copy(ones_v, deg2d.at[didx.at[j]], dsem, add=True)
        for j in range(NB):
            pltpu.make_async_copy(ones_v, deg2d.at[didx.at[j]], dsem).wait()
        return 0
    lax.fori_loop(0, NBLK, _blk, 0)

    plsc.subcore_barrier()

    @pl.when(c == 0)
    def _():
        pltpu.sync_copy(deg2d.at[pl.ds(base, RPT)],
                        deg_out.at[pl.ds(base, RPT)])


_SC_MESH = plsc.VectorSubcoreMesh(core_axis_name="c", subcore_axis_name="s")

_sc_agg = pl.kernel(
    _sc_agg_body,
    out_type=[jax.ShapeDtypeStruct((N2, 2, HALF), _f32)],
    mesh=_SC_MESH,
    scratch_types=[
        pltpu.VMEM_SHARED((N2, HALF), _f32),
        pltpu.VMEM((NB, CH), jnp.int32),
        pltpu.VMEM((NB, CH), jnp.int32),
        pltpu.VMEM((CH, HALF), _f32),
        pltpu.VMEM((CH, HALF), _f32),
        pltpu.SemaphoreType.DMA,
        pltpu.SemaphoreType.DMA,
    ],
)

_sc_deg = pl.kernel(
    _sc_deg_body,
    out_type=[jax.ShapeDtypeStruct((N2, HALF), _f32)],
    mesh=_SC_MESH,
    scratch_types=[
        pltpu.VMEM_SHARED((N2, HALF), _f32),
        pltpu.VMEM((NB, CH), jnp.int32),
        pltpu.VMEM((CH, HALF), _f32),
        pltpu.SemaphoreType.DMA,
    ],
)


BR1 = 512
BR3 = 400


def _tc1_body(a, deg, x, wl, wr, b, out):
    inv = 1.0 / jnp.maximum(deg[:, 0:1], 1.0)
    acc = jnp.dot(a[:, 0, :] * inv, wl[:HALF], preferred_element_type=_f32)
    acc += jnp.dot(a[:, 1, :] * inv, wl[HALF:], preferred_element_type=_f32)
    acc += jnp.dot(x[...], wr[...], preferred_element_type=_f32)
    out[...] = jnp.maximum(acc + b[...], 0.0)


def _tc1(agg, deg, x_p, w1l, w1r, b1):
    return pl.pallas_call(
        _tc1_body,
        grid=(N2 // BR1,),
        in_specs=[
            pl.BlockSpec((BR1, 2, HALF), lambda i: (i, 0, 0)),
            pl.BlockSpec((BR1, HALF), lambda i: (i, 0)),
            pl.BlockSpec((BR1, DIN), lambda i: (i, 0)),
            pl.BlockSpec((DIN, DHID), lambda i: (0, 0)),
            pl.BlockSpec((DIN, DHID), lambda i: (0, 0)),
            pl.BlockSpec((1, DHID), lambda i: (0, 0)),
        ],
        out_specs=pl.BlockSpec((BR1, DHID), lambda i: (i, 0)),
        out_shape=jax.ShapeDtypeStruct((N2, DHID), _f32),
    )(agg, deg, x_p, w1l, w1r, b1)


def _tc2_body(h, wl, wr, b, p_out, r_out):
    hh = h[...]
    p = jnp.dot(hh, wl[...], preferred_element_type=_f32)
    r = jnp.dot(hh, wr[...], preferred_element_type=_f32) + b[...]
    p_out[:, 0, :] = p[:, :HALF]
    p_out[:, 1, :] = p[:, HALF:]
    r_out[:, 0, :] = r[:, :HALF]
    r_out[:, 1, :] = r[:, HALF:]


def _tc2(h, w2l, w2r, b2):
    return pl.pallas_call(
        _tc2_body,
        grid=(N2 // BR1,),
        in_specs=[
            pl.BlockSpec((BR1, DHID), lambda i: (i, 0)),
            pl.BlockSpec((DHID, DOUT), lambda i: (0, 0)),
            pl.BlockSpec((DHID, DOUT), lambda i: (0, 0)),
            pl.BlockSpec((1, DOUT), lambda i: (0, 0)),
        ],
        out_specs=[
            pl.BlockSpec((BR1, 2, HALF), lambda i: (i, 0, 0)),
            pl.BlockSpec((BR1, 2, HALF), lambda i: (i, 0, 0)),
        ],
        out_shape=[
            jax.ShapeDtypeStruct((N2, 2, HALF), _f32),
            jax.ShapeDtypeStruct((N2, 2, HALF), _f32),
        ],
    )(h, w2l, w2r, b2)


def _tc3_body(a, deg, r, z):
    inv = 1.0 / jnp.maximum(deg[:, 0:1], 1.0)
    z[...] = jnp.concatenate(
        [a[:, 0, :] * inv + r[:, 0, :], a[:, 1, :] * inv + r[:, 1, :]],
        axis=1)


def _tc3(agg2, deg, r3):
    return pl.pallas_call(
        _tc3_body,
        grid=(N // BR3,),
        in_specs=[
            pl.BlockSpec((BR3, 2, HALF), lambda i: (i, 0, 0)),
            pl.BlockSpec((BR3, HALF), lambda i: (i, 0)),
            pl.BlockSpec((BR3, 2, HALF), lambda i: (i, 0, 0)),
        ],
        out_specs=pl.BlockSpec((BR3, DOUT), lambda i: (i, 0)),
        out_shape=jax.ShapeDtypeStruct((N, DOUT), _f32),
    )(agg2, deg, r3)


def kernel(x, edge_index, W1_l, W1_r, b1, W2_l, W2_r, b2):
    src = edge_index[0].astype(jnp.int32)
    dst = edge_index[1].astype(jnp.int32)
    padi = (jnp.arange(E2 - E, dtype=jnp.int32) % 16) + N
    srcs = jnp.concatenate([src, padi]).reshape(NT, NCHUNK, CH)
    dsts = jnp.concatenate([dst, padi]).reshape(NT, NCHUNK, CH)

    x_p = jnp.pad(x, ((0, N2 - N), (0, 0)))
    xflat = x_p.reshape(2 * N2, HALF)

    (deg,) = _sc_deg(dsts)
    (agg1,) = _sc_agg(xflat, srcs, dsts)
    h = _tc1(agg1, deg, x_p, W1_l, W1_r, b1.reshape(1, DHID))
    p3, r3 = _tc2(h, W2_l, W2_r, b2.reshape(1, DOUT))
    (agg2,) = _sc_agg(p3.reshape(2 * N2, HALF), srcs, dsts)
    return _tc3(agg2, deg, r3)


# speedup vs baseline: 7.4653x; 1.3054x over previous
import jax
import jax.numpy as jnp
from jax import lax
from jax.experimental import pallas as pl
from jax.experimental.pallas import tpu as pltpu
from jax.experimental.pallas import tpu_sc as plsc

N = 10000
N2 = 10240
E = 160000
E2 = 163840
DIN = 256
DHID = 512
DOUT = 256
HALF = 128
NT = 16
_f32 = jnp.float32

CH = 128
NB = 8
NCHUNK = E2 // (NT * CH)
NBLK = NCHUNK // NB
RPT = N2 // NT


def _sc_agg_body(table, srcs, dsts, agg_out, accum, sidx, didx,
                 gbuf0, gbuf1, gsem0, gsem1, isem):
    c = lax.axis_index("c")
    s = lax.axis_index("s")
    base = s * RPT

    def _zrow(i, _):
        for k in range(HALF // 16):
            gbuf0[i, pl.ds(k * 16, 16)] = jnp.zeros((16,), _f32)
        return 0
    lax.fori_loop(0, CH, _zrow, 0)
    for q in range(RPT // CH):
        pltpu.sync_copy(gbuf0, accum.at[pl.ds(base + q * CH, CH)])

    plsc.subcore_barrier()

    def _idx_start(b, par):
        pltpu.async_copy(srcs.at[s, pl.ds(b * NB, NB)], sidx.at[par], isem)
        pltpu.async_copy(dsts.at[s, pl.ds(b * NB, NB)], didx.at[par], isem)

    def _idx_wait(b, par):
        pltpu.make_async_copy(srcs.at[s, pl.ds(b * NB, NB)], sidx.at[par],
                              isem).wait()
        pltpu.make_async_copy(dsts.at[s, pl.ds(b * NB, NB)], didx.at[par],
                              isem).wait()

    _idx_start(0, 0)

    def _blk(b, _):
        par = lax.rem(b, 2)
        _idx_wait(b, par)

        @pl.when(b + 1 < NBLK)
        def _():
            _idx_start(b + 1, 1 - par)

        si = sidx.at[par]
        di = didx.at[par]

        def _fix(j, _):
            for k in range(CH // 16):
                sl = si[j, pl.ds(k * 16, 16)]
                si[j, pl.ds(k * 16, 16)] = sl * 2 + c
            return 0
        lax.fori_loop(0, NB, _fix, 0)

        bufs = (gbuf0, gbuf1)
        sems = (gsem0, gsem1)
        pltpu.async_copy(table.at[si.at[0]], bufs[0], sems[0])
        for j in range(NB):
            bj, sj = bufs[j % 2], sems[j % 2]
            pltpu.make_async_copy(table.at[si.at[j]], bj, sj).wait()
            if j + 1 < NB:
                pltpu.async_copy(table.at[si.at[j + 1]],
                                 bufs[(j + 1) % 2], sems[(j + 1) % 2])
            pltpu.sync_copy(bj, accum.at[di.at[j]], add=True)
        return 0
    lax.fori_loop(0, NBLK, _blk, 0)

    plsc.subcore_barrier()

    pltpu.sync_copy(accum.at[pl.ds(base, RPT)],
                    agg_out.at[pl.ds(base, RPT), c])


def _sc_deg_body(dsts, deg_out, deg2d, didx, ones_v, dsem):
    c = lax.axis_index("c")
    s = lax.axis_index("s")
    base = s * RPT

    def _zrow(i, _):
        for k in range(HALF // 16):
            ones_v[i, pl.ds(k * 16, 16)] = jnp.zeros((16,), _f32)
        return 0
    lax.fori_loop(0, CH, _zrow, 0)
    for q in range(RPT // CH):
        pltpu.sync_copy(ones_v, deg2d.at[pl.ds(base + q * CH, CH)])

    def _orow(i, _):
        for k in range(HALF // 16):
            ones_v[i, pl.ds(k * 16, 16)] = jnp.full((16,), 1.0, _f32)
        return 0
    lax.fori_loop(0, CH, _orow, 0)

    plsc.subcore_barrier()

    def _blk(b, _):
        pltpu.sync_copy(dsts.at[s, pl.ds(b * NB, NB)], didx)
        for j in range(NB):
            pltpu.async_copy(ones_v, deg2d.at[didx.at[j]], dsem, add=True)
        for j in range(NB):
            pltpu.make_async_copy(ones_v, deg2d.at[didx.at[j]], dsem).wait()
        return 0
    lax.fori_loop(0, NBLK, _blk, 0)

    plsc.subcore_barrier()

    @pl.when(c == 0)
    def _():
        pltpu.sync_copy(deg2d.at[pl.ds(base, RPT)],
                        deg_out.at[pl.ds(base, RPT)])


_SC_MESH = plsc.VectorSubcoreMesh(core_axis_name="c", subcore_axis_name="s")

_sc_agg = pl.kernel(
    _sc_agg_body,
    out_type=[jax.ShapeDtypeStruct((N2, 2, HALF), _f32)],
    mesh=_SC_MESH,
    scratch_types=[
        pltpu.VMEM_SHARED((N2, HALF), _f32),
        pltpu.VMEM((2, NB, CH), jnp.int32),
        pltpu.VMEM((2, NB, CH), jnp.int32),
        pltpu.VMEM((CH, HALF), _f32),
        pltpu.VMEM((CH, HALF), _f32),
        pltpu.SemaphoreType.DMA,
        pltpu.SemaphoreType.DMA,
        pltpu.SemaphoreType.DMA,
    ],
)

_sc_deg = pl.kernel(
    _sc_deg_body,
    out_type=[jax.ShapeDtypeStruct((N2, HALF), _f32)],
    mesh=_SC_MESH,
    scratch_types=[
        pltpu.VMEM_SHARED((N2, HALF), _f32),
        pltpu.VMEM((NB, CH), jnp.int32),
        pltpu.VMEM((CH, HALF), _f32),
        pltpu.SemaphoreType.DMA,
    ],
)


BR1 = 512
BR3 = 400


def _tc1_body(a, deg, x, wl, wr, b, out):
    inv = 1.0 / jnp.maximum(deg[:, 0:1], 1.0)
    acc = jnp.dot(a[:, 0, :] * inv, wl[:HALF], preferred_element_type=_f32)
    acc += jnp.dot(a[:, 1, :] * inv, wl[HALF:], preferred_element_type=_f32)
    acc += jnp.dot(x[...], wr[...], preferred_element_type=_f32)
    out[...] = jnp.maximum(acc + b[...], 0.0)


def _tc1(agg, deg, x_p, w1l, w1r, b1):
    return pl.pallas_call(
        _tc1_body,
        grid=(N2 // BR1,),
        in_specs=[
            pl.BlockSpec((BR1, 2, HALF), lambda i: (i, 0, 0)),
            pl.BlockSpec((BR1, HALF), lambda i: (i, 0)),
            pl.BlockSpec((BR1, DIN), lambda i: (i, 0)),
            pl.BlockSpec((DIN, DHID), lambda i: (0, 0)),
            pl.BlockSpec((DIN, DHID), lambda i: (0, 0)),
            pl.BlockSpec((1, DHID), lambda i: (0, 0)),
        ],
        out_specs=pl.BlockSpec((BR1, DHID), lambda i: (i, 0)),
        out_shape=jax.ShapeDtypeStruct((N2, DHID), _f32),
    )(agg, deg, x_p, w1l, w1r, b1)


def _tc2_body(h, wl, wr, b, p_out, r_out):
    hh = h[...]
    p = jnp.dot(hh, wl[...], preferred_element_type=_f32)
    r = jnp.dot(hh, wr[...], preferred_element_type=_f32) + b[...]
    p_out[:, 0, :] = p[:, :HALF]
    p_out[:, 1, :] = p[:, HALF:]
    r_out[:, 0, :] = r[:, :HALF]
    r_out[:, 1, :] = r[:, HALF:]


def _tc2(h, w2l, w2r, b2):
    return pl.pallas_call(
        _tc2_body,
        grid=(N2 // BR1,),
        in_specs=[
            pl.BlockSpec((BR1, DHID), lambda i: (i, 0)),
            pl.BlockSpec((DHID, DOUT), lambda i: (0, 0)),
            pl.BlockSpec((DHID, DOUT), lambda i: (0, 0)),
            pl.BlockSpec((1, DOUT), lambda i: (0, 0)),
        ],
        out_specs=[
            pl.BlockSpec((BR1, 2, HALF), lambda i: (i, 0, 0)),
            pl.BlockSpec((BR1, 2, HALF), lambda i: (i, 0, 0)),
        ],
        out_shape=[
            jax.ShapeDtypeStruct((N2, 2, HALF), _f32),
            jax.ShapeDtypeStruct((N2, 2, HALF), _f32),
        ],
    )(h, w2l, w2r, b2)


def _tc3_body(a, deg, r, z):
    inv = 1.0 / jnp.maximum(deg[:, 0:1], 1.0)
    z[...] = jnp.concatenate(
        [a[:, 0, :] * inv + r[:, 0, :], a[:, 1, :] * inv + r[:, 1, :]],
        axis=1)


def _tc3(agg2, deg, r3):
    return pl.pallas_call(
        _tc3_body,
        grid=(N // BR3,),
        in_specs=[
            pl.BlockSpec((BR3, 2, HALF), lambda i: (i, 0, 0)),
            pl.BlockSpec((BR3, HALF), lambda i: (i, 0)),
            pl.BlockSpec((BR3, 2, HALF), lambda i: (i, 0, 0)),
        ],
        out_specs=pl.BlockSpec((BR3, DOUT), lambda i: (i, 0)),
        out_shape=jax.ShapeDtypeStruct((N, DOUT), _f32),
    )(agg2, deg, r3)


def kernel(x, edge_index, W1_l, W1_r, b1, W2_l, W2_r, b2):
    src = edge_index[0].astype(jnp.int32)
    dst = edge_index[1].astype(jnp.int32)
    padi = (jnp.arange(E2 - E, dtype=jnp.int32) % 16) + N
    srcs = jnp.concatenate([src, padi]).reshape(NT, NCHUNK, CH)
    dsts = jnp.concatenate([dst, padi]).reshape(NT, NCHUNK, CH)

    x_p = jnp.pad(x, ((0, N2 - N), (0, 0)))
    xflat = x_p.reshape(2 * N2, HALF)

    (deg,) = _sc_deg(dsts)
    (agg1,) = _sc_agg(xflat, srcs, dsts)
    h = _tc1(agg1, deg, x_p, W1_l, W1_r, b1.reshape(1, DHID))
    p3, r3 = _tc2(h, W2_l, W2_r, b2.reshape(1, DOUT))
    (agg2,) = _sc_agg(p3.reshape(2 * N2, HALF), srcs, dsts)
    return _tc3(agg2, deg, r3)
